# baseline (speedup 1.0000x reference)
import functools

import jax
import jax.numpy as jnp
from jax import lax
from jax.experimental import pallas as pl
from jax.experimental.pallas import tpu as pltpu
from jax.experimental.pallas import tpu_sc as plsc

NC = 2
NS = 16
NW = NC * NS
L = 16
CHUNK = 128


def _sc_body(nch, d, idx_hbm, table_hbm, out_hbm, idx_v, rows_v, out_v, sem):
    bpw = nch * CHUNK
    wid = lax.axis_index("s") * NC + lax.axis_index("c")
    base = wid * bpw

    pltpu.sync_copy(idx_hbm.at[wid], idx_v)
    copies = [
        pltpu.async_copy(
            table_hbm.at[idx_v.at[j]],
            rows_v.at[pl.ds(j * CHUNK, CHUNK)],
            sem,
        )
        for j in range(nch)
    ]
    for c in copies:
        c.wait()

    lane = jnp.arange(L, dtype=jnp.int32)

    def row_body(r, carry):
        rvec = jnp.full((L,), 0, jnp.int32) + r
        for j in range(d // L):
            e = rows_v[r, pl.ds(j * L, L)]
            p = 1.0 / (1.0 + jnp.exp(-e))
            ceven = 2 * lane + (2 * j * L)
            plsc.store_scatter(out_v, [rvec, ceven], p)
            plsc.store_scatter(out_v, [rvec, ceven + 1], 1.0 - p)
        return carry

    lax.fori_loop(0, bpw, row_body, 0)

    pltpu.sync_copy(out_v, out_hbm.at[pl.ds(base, bpw)])


def _sc_lookup(idx_w, table):
    nw, nch, _ = idx_w.shape
    d = table.shape[1]
    bpw = nch * CHUNK
    b = nw * bpw
    mesh = plsc.VectorSubcoreMesh(core_axis_name="c", subcore_axis_name="s")
    return pl.kernel(
        functools.partial(_sc_body, nch, d),
        out_type=jax.ShapeDtypeStruct((b, 2 * d), jnp.float32),
        mesh=mesh,
        scratch_types=[
            pltpu.VMEM((nch, CHUNK), jnp.int32),
            pltpu.VMEM((bpw, d), jnp.float32),
            pltpu.VMEM((bpw, 2 * d), jnp.float32),
            pltpu.SemaphoreType.DMA,
        ],
        compiler_params=pltpu.CompilerParams(
            use_tc_tiling_on_sc=False, needs_layout_passes=False
        ),
    )(idx_w, table)


def kernel(idx, embeddings):
    b = idx.shape[0]
    d = embeddings.shape[1]
    idx_w = idx.astype(jnp.int32).reshape(NW, b // (NW * CHUNK), CHUNK)
    out = _sc_lookup(idx_w, embeddings)
    return out.reshape(b, d, 2)


# speedup vs baseline: 1.6150x; 1.6150x over previous
import functools

import jax
import jax.numpy as jnp
from jax import lax
from jax.experimental import pallas as pl
from jax.experimental.pallas import tpu as pltpu
from jax.experimental.pallas import tpu_sc as plsc

NC = 2
NS = 16
NW = NC * NS
L = 16
DEPTH = 32


def _sc_body(bpw, d, idx_hbm, table_hbm, out_hbm, idx_v, idx_s, rows_v, out_v,
             sem):
    wid = lax.axis_index("s") * NC + lax.axis_index("c")

    pltpu.sync_copy(idx_hbm.at[wid], idx_v)

    gdepth = DEPTH // L
    ngroups = bpw // L

    def fire(g, carry):
        v = idx_v[pl.ds(g * L, L)]
        for l in range(L):
            pltpu.async_copy(
                table_hbm.at[pl.ds(v[l], 1)],
                rows_v.at[pl.ds(g * L + l, 1)],
                sem,
            )
        return carry

    def drain(g, carry):
        for l in range(L):
            pltpu.make_async_copy(
                table_hbm.at[pl.ds(0, 1)],
                rows_v.at[pl.ds(g * L + l, 1)],
                sem,
            ).wait()
        return carry

    def fire_drain(g, carry):
        fire(g, carry)
        return drain(g - gdepth, carry)

    lax.fori_loop(0, gdepth, fire, 0)
    lax.fori_loop(gdepth, ngroups, fire_drain, 0)
    lax.fori_loop(ngroups - gdepth, ngroups, drain, 0)

    lane = jnp.arange(L, dtype=jnp.int32)
    half_rows = bpw // 2

    for half in range(2):
        def row_body(rl, carry):
            r = half * half_rows + rl
            rvec = jnp.full((L,), 0, jnp.int32) + rl
            for j in range(d // L):
                e = rows_v[r, pl.ds(j * L, L)]
                p = 1.0 / (1.0 + jnp.exp(-e))
                ceven = 2 * L * j + 2 * lane
                plsc.store_scatter(out_v, [rvec, ceven], p)
                plsc.store_scatter(out_v, [rvec, ceven + 1], 1.0 - p)
            return carry

        lax.fori_loop(0, half_rows, row_body, 0)
        pltpu.sync_copy(
            out_v,
            out_hbm.at[pl.ds(wid * bpw + half * half_rows, half_rows)],
        )


def _sc_lookup(idx_w, table):
    nw, bpw = idx_w.shape
    d = table.shape[1]
    mesh = plsc.VectorSubcoreMesh(core_axis_name="c", subcore_axis_name="s")
    return pl.kernel(
        functools.partial(_sc_body, bpw, d),
        out_type=jax.ShapeDtypeStruct((nw * bpw, 2 * d), jnp.float32),
        mesh=mesh,
        scratch_types=[
            pltpu.VMEM((bpw,), jnp.int32),
            pltpu.SMEM((bpw,), jnp.int32),
            pltpu.VMEM((bpw, d), jnp.float32),
            pltpu.VMEM((bpw // 2, 2 * d), jnp.float32),
            pltpu.SemaphoreType.DMA,
        ],
        compiler_params=pltpu.CompilerParams(
            use_tc_tiling_on_sc=True, needs_layout_passes=False
        ),
    )(idx_w, table)


def kernel(idx, embeddings):
    b = idx.shape[0]
    d = embeddings.shape[1]
    idx_w = idx.astype(jnp.int32).reshape(NW, b // NW)
    out = _sc_lookup(idx_w, embeddings)
    return out.reshape(b, d, 2)
